# Initial kernel scaffold; baseline (speedup 1.0000x reference)
import jax
import jax.numpy as jnp
from jax import lax
from jax.experimental import pallas as pl
from jax.experimental.pallas import tpu as pltpu
from jax.experimental.pallas import tpu_sc as plsc

N_V = 10000
N_E = 20000
NNZ = 320000
D = 128
DH = 64
NS = 16
IB = 125
NROW = NNZ // IB
R_T = NROW // NS
E_PAD = 20480
V_PAD = 10240
E_T = E_PAD // NS
V_T = V_PAD // NS
ER_T = N_E // NS
VR_T = N_V // NS

_f32 = jnp.float32
_i32 = jnp.int32


def _zeros16():
    return jnp.zeros((16,), _f32)


def _counts_body(vidx, eidx, rce_hbm, rcv_hbm, idx_buf, ones_buf, dbuf, pk,
                 cnt_sh):
    c = lax.axis_index("c")
    s = lax.axis_index("s")

    def _z(i, carry):
        dbuf[i] = _zeros16()
        return carry
    lax.fori_loop(0, E_T, _z, 0)
    pltpu.sync_copy(dbuf, cnt_sh.at[pl.ds(s * E_T, E_T)])

    def _o(i, carry):
        ones_buf[i] = jnp.ones((16,), _f32)
        return carry
    lax.fori_loop(0, IB, _o, 0)

    @pl.when(c == 0)
    def _():
        pltpu.sync_copy(eidx.at[pl.ds(s * R_T, R_T)], idx_buf)

    @pl.when(c == 1)
    def _():
        pltpu.sync_copy(vidx.at[pl.ds(s * R_T, R_T)], idx_buf)

    plsc.subcore_barrier()

    def _sc(j, carry):
        pltpu.sync_copy(ones_buf, cnt_sh.at[idx_buf.at[j]], add=True)
        return carry
    lax.fori_loop(0, R_T, _sc, 0)

    plsc.subcore_barrier()

    zeros_i = jnp.zeros((16,), _i32)
    iota16 = lax.iota(_i32, 16)

    def _extract(nrows, base, out_ref):
        pltpu.sync_copy(cnt_sh.at[pl.ds(base, nrows)], dbuf.at[pl.ds(0, nrows)])

        def _g(g, carry):
            rows16 = iota16 + g * 16
            cnt = plsc.load_gather(dbuf, [rows16, zeros_i])
            pk[pl.ds(g * 16, 16)] = 1.0 / jnp.maximum(cnt, 1.0)
            return carry
        lax.fori_loop(0, nrows // 16, _g, 0)
        pltpu.sync_copy(pk.at[pl.ds(0, nrows)], out_ref.at[pl.ds(base, nrows)])

    @pl.when(c == 0)
    def _():
        _extract(E_T, s * E_T, rce_hbm)

    @pl.when(c == 1)
    def _():
        _extract(V_T, s * V_T, rcv_hbm)


@jax.jit
def _counts(vidx2, eidx2):
    return pl.kernel(
        _counts_body,
        out_type=(
            jax.ShapeDtypeStruct((E_PAD,), _f32),
            jax.ShapeDtypeStruct((V_PAD,), _f32),
        ),
        mesh=plsc.VectorSubcoreMesh(core_axis_name="c", subcore_axis_name="s"),
        scratch_types=[
            pltpu.VMEM((R_T, IB), _i32),
            pltpu.VMEM((IB, 16), _f32),
            pltpu.VMEM((E_T, 16), _f32),
            pltpu.VMEM((E_T,), _f32),
            pltpu.VMEM_SHARED((E_PAD, 16), _f32),
        ],
    )(vidx2, eidx2)


def _v2v_body(vidx, eidx, h0, h1, rce, m0, m1,
              vbuf, ebuf, rows, rbuf, dbuf, e_acc, v_acc):
    c = lax.axis_index("c")
    s = lax.axis_index("s")

    def _z(i, carry):
        for k in range(DH // 16):
            rows[i, pl.ds(k * 16, 16)] = _zeros16()
        return carry
    lax.fori_loop(0, IB, _z, 0)

    def _ze(i, carry):
        pltpu.sync_copy(rows, e_acc.at[pl.ds(s * ER_T + i * IB, IB)])
        return carry
    lax.fori_loop(0, ER_T // IB, _ze, 0)

    def _zv(i, carry):
        pltpu.sync_copy(rows, v_acc.at[pl.ds(s * VR_T + i * IB, IB)])
        return carry
    lax.fori_loop(0, VR_T // IB, _zv, 0)

    pltpu.sync_copy(vidx.at[pl.ds(s * R_T, R_T)], vbuf)
    pltpu.sync_copy(eidx.at[pl.ds(s * R_T, R_T)], ebuf)
    pltpu.sync_copy(rce, rbuf)
    plsc.subcore_barrier()

    def _p1(h_ref):
        def step(j, carry):
            pltpu.sync_copy(h_ref.at[vbuf.at[j]], rows)
            pltpu.sync_copy(rows, e_acc.at[ebuf.at[j]], add=True)
            return carry
        lax.fori_loop(0, R_T, step, 0)

    @pl.when(c == 0)
    def _():
        _p1(h0)

    @pl.when(c == 1)
    def _():
        _p1(h1)

    plsc.subcore_barrier()

    def _p2(i, carry):
        base = s * ER_T + i * IB
        pltpu.sync_copy(e_acc.at[pl.ds(base, IB)], dbuf)

        def rowstep(r, carry2):
            idx = jnp.broadcast_to(base + r, (16,)).astype(_i32)
            spl = plsc.load_gather(rbuf, [idx])
            for k in range(DH // 16):
                dbuf[r, pl.ds(k * 16, 16)] = dbuf[r, pl.ds(k * 16, 16)] * spl
            return carry2
        lax.fori_loop(0, IB, rowstep, 0)
        pltpu.sync_copy(dbuf, e_acc.at[pl.ds(base, IB)])
        return carry
    lax.fori_loop(0, ER_T // IB, _p2, 0)
    plsc.subcore_barrier()

    def _p3(j, carry):
        pltpu.sync_copy(e_acc.at[ebuf.at[j]], rows)
        pltpu.sync_copy(rows, v_acc.at[vbuf.at[j]], add=True)
        return carry
    lax.fori_loop(0, R_T, _p3, 0)
    plsc.subcore_barrier()

    @pl.when(c == 0)
    def _():
        pltpu.sync_copy(v_acc.at[pl.ds(s * VR_T, VR_T)],
                        m0.at[pl.ds(s * VR_T, VR_T)])

    @pl.when(c == 1)
    def _():
        pltpu.sync_copy(v_acc.at[pl.ds(s * VR_T, VR_T)],
                        m1.at[pl.ds(s * VR_T, VR_T)])


@jax.jit
def _v2v(vidx2, eidx2, h0, h1, rce):
    return pl.kernel(
        _v2v_body,
        out_type=(
            jax.ShapeDtypeStruct((N_V, DH), _f32),
            jax.ShapeDtypeStruct((N_V, DH), _f32),
        ),
        mesh=plsc.VectorSubcoreMesh(core_axis_name="c", subcore_axis_name="s"),
        scratch_types=[
            pltpu.VMEM((R_T, IB), _i32),
            pltpu.VMEM((R_T, IB), _i32),
            pltpu.VMEM((IB, DH), _f32),
            pltpu.VMEM((E_PAD,), _f32),
            pltpu.VMEM((IB, DH), _f32),
            pltpu.VMEM_SHARED((N_E, DH), _f32),
            pltpu.VMEM_SHARED((N_V, DH), _f32),
        ],
    )(vidx2, eidx2, h0, h1, rce)


_BM = 1000


def _mm1_body(x_ref, w_ref, b_ref, o0_ref, o1_ref):
    h = jnp.dot(x_ref[...], w_ref[...], preferred_element_type=_f32)
    h = jnp.maximum(h + b_ref[...], 0.0)
    o0_ref[...] = h[:, :DH]
    o1_ref[...] = h[:, DH:]


@jax.jit
def _mm1(x, w1, b1):
    return pl.pallas_call(
        _mm1_body,
        grid=(N_V // _BM,),
        in_specs=[
            pl.BlockSpec((_BM, D), lambda i: (i, 0)),
            pl.BlockSpec((D, D), lambda i: (0, 0)),
            pl.BlockSpec((1, D), lambda i: (0, 0)),
        ],
        out_specs=[
            pl.BlockSpec((_BM, DH), lambda i: (i, 0)),
            pl.BlockSpec((_BM, DH), lambda i: (i, 0)),
        ],
        out_shape=[
            jax.ShapeDtypeStruct((N_V, DH), _f32),
            jax.ShapeDtypeStruct((N_V, DH), _f32),
        ],
    )(x, w1, b1)


def _mm2_body(m0_ref, m1_ref, rv_ref, w_ref, b_ref, o0_ref, o1_ref):
    x = jnp.concatenate([m0_ref[...], m1_ref[...]], axis=1)
    x = jnp.maximum(x, 0.0) * rv_ref[...]
    h = jnp.dot(x, w_ref[...], preferred_element_type=_f32)
    h = jnp.maximum(h + b_ref[...], 0.0)
    o0_ref[...] = h[:, :DH]
    o1_ref[...] = h[:, DH:]


@jax.jit
def _mm2(m0, m1, rv, w2, b2):
    return pl.pallas_call(
        _mm2_body,
        grid=(N_V // _BM,),
        in_specs=[
            pl.BlockSpec((_BM, DH), lambda i: (i, 0)),
            pl.BlockSpec((_BM, DH), lambda i: (i, 0)),
            pl.BlockSpec((_BM, 1), lambda i: (i, 0)),
            pl.BlockSpec((D, D), lambda i: (0, 0)),
            pl.BlockSpec((1, D), lambda i: (0, 0)),
        ],
        out_specs=[
            pl.BlockSpec((_BM, DH), lambda i: (i, 0)),
            pl.BlockSpec((_BM, DH), lambda i: (i, 0)),
        ],
        out_shape=[
            jax.ShapeDtypeStruct((N_V, DH), _f32),
            jax.ShapeDtypeStruct((N_V, DH), _f32),
        ],
    )(m0, m1, rv, w2, b2)


def _merge_body(m0_ref, m1_ref, rv_ref, o_ref):
    x = jnp.concatenate([m0_ref[...], m1_ref[...]], axis=1)
    o_ref[...] = x * rv_ref[...]


@jax.jit
def _merge(m0, m1, rv):
    return pl.pallas_call(
        _merge_body,
        grid=(N_V // _BM,),
        in_specs=[
            pl.BlockSpec((_BM, DH), lambda i: (i, 0)),
            pl.BlockSpec((_BM, DH), lambda i: (i, 0)),
            pl.BlockSpec((_BM, 1), lambda i: (i, 0)),
        ],
        out_specs=pl.BlockSpec((_BM, D), lambda i: (i, 0)),
        out_shape=jax.ShapeDtypeStruct((N_V, D), _f32),
    )(m0, m1, rv)


def kernel(X, v_idx, e_idx, W1, b1, W2, b2):
    vidx2 = v_idx.astype(_i32).reshape(NROW, IB)
    eidx2 = e_idx.astype(_i32).reshape(NROW, IB)
    b1r = b1.reshape(1, D)
    b2r = b2.reshape(1, D)

    rce, rcv = _counts(vidx2, eidx2)
    rv = rcv[:N_V].reshape(N_V, 1)

    h0, h1 = _mm1(X, W1, b1r)
    m0, m1 = _v2v(vidx2, eidx2, h0, h1, rce)
    g0, g1 = _mm2(m0, m1, rv, W2, b2r)
    n0, n1 = _v2v(vidx2, eidx2, g0, g1, rce)
    return _merge(n0, n1, rv)


# speedup vs baseline: 5.9187x; 5.9187x over previous
import jax
import jax.numpy as jnp
from jax import lax
from jax.experimental import pallas as pl
from jax.experimental.pallas import tpu as pltpu
from jax.experimental.pallas import tpu_sc as plsc

N_V = 10000
N_E = 20000
NNZ = 320000
D = 128
DH = 64
NS = 16
IB = 125
NROW = NNZ // IB
R_T = NROW // NS
E_PAD = 20480
V_PAD = 10240
E_T = E_PAD // NS
V_T = V_PAD // NS
ER_T = N_E // NS
VR_T = N_V // NS
CH = 32

_f32 = jnp.float32
_i32 = jnp.int32


def _zeros16():
    return jnp.zeros((16,), _f32)


def _counts_body(vidx, eidx, rce_hbm, rcv_hbm, idx_buf, ones_buf, dbuf,
                 cnt_sh):
    c = lax.axis_index("c")
    s = lax.axis_index("s")

    def _z(i, carry):
        dbuf[i] = _zeros16()
        return carry
    lax.fori_loop(0, E_T, _z, 0)
    pltpu.sync_copy(dbuf, cnt_sh.at[pl.ds(s * E_T, E_T)])

    def _o(i, carry):
        ones_buf[i] = jnp.ones((16,), _f32)
        return carry
    lax.fori_loop(0, IB, _o, 0)

    @pl.when(c == 0)
    def _():
        pltpu.sync_copy(eidx.at[pl.ds(s * R_T, R_T)], idx_buf)

    @pl.when(c == 1)
    def _():
        pltpu.sync_copy(vidx.at[pl.ds(s * R_T, R_T)], idx_buf)

    plsc.subcore_barrier()

    def _sc(j, carry):
        pltpu.sync_copy(ones_buf, cnt_sh.at[idx_buf.at[j]], add=True)
        return carry
    lax.fori_loop(0, R_T, _sc, 0)

    plsc.subcore_barrier()

    def _extract(nrows, base, out_ref):
        pltpu.sync_copy(cnt_sh.at[pl.ds(base, nrows)], dbuf.at[pl.ds(0, nrows)])

        def _r(r, carry):
            dbuf[r] = 1.0 / jnp.maximum(dbuf[r], 1.0)
            return carry
        lax.fori_loop(0, nrows, _r, 0)
        pltpu.sync_copy(dbuf.at[pl.ds(0, nrows)], out_ref.at[pl.ds(base, nrows)])

    @pl.when(c == 0)
    def _():
        _extract(E_T, s * E_T, rce_hbm)

    @pl.when(c == 1)
    def _():
        _extract(V_T, s * V_T, rcv_hbm)


@jax.jit
def _counts(vidx2, eidx2):
    return pl.kernel(
        _counts_body,
        out_type=(
            jax.ShapeDtypeStruct((E_PAD, 16), _f32),
            jax.ShapeDtypeStruct((V_PAD, 16), _f32),
        ),
        mesh=plsc.VectorSubcoreMesh(core_axis_name="c", subcore_axis_name="s"),
        compiler_params=pltpu.CompilerParams(use_tc_tiling_on_sc=False),
        scratch_types=[
            pltpu.VMEM((R_T, IB), _i32),
            pltpu.VMEM((IB, 16), _f32),
            pltpu.VMEM((E_T, 16), _f32),
            pltpu.VMEM_SHARED((E_PAD, 16), _f32),
        ],
    )(vidx2, eidx2)


def _v2v_body(vidx, eidx, h0, h1, rce, m0, m1, xe0, xe1,
              vbuf, ebuf, rows, rbuf, dbuf, acc):
    c = lax.axis_index("c")
    s = lax.axis_index("s")

    def _zero_rows():
        def _z(i, carry):
            for k in range(DH // 16):
                rows[i, pl.ds(k * 16, 16)] = _zeros16()
            return carry
        lax.fori_loop(0, IB, _z, 0)

    _zero_rows()

    def _ze(i, carry):
        pltpu.sync_copy(rows, acc.at[pl.ds(s * ER_T + i * IB, IB)])
        return carry
    lax.fori_loop(0, ER_T // IB, _ze, 0)
    plsc.subcore_barrier()

    def _p1(h_ref):
        def grp(g, carry):
            base = s * R_T + g * CH
            pltpu.sync_copy(vidx.at[pl.ds(base, CH)], vbuf)
            pltpu.sync_copy(eidx.at[pl.ds(base, CH)], ebuf)

            def step(j, carry2):
                pltpu.sync_copy(h_ref.at[vbuf.at[j]], rows)
                pltpu.sync_copy(rows, acc.at[ebuf.at[j]], add=True)
                return carry2
            lax.fori_loop(0, CH, step, 0)
            return carry
        lax.fori_loop(0, R_T // CH, grp, 0)

    @pl.when(c == 0)
    def _():
        _p1(h0)

    @pl.when(c == 1)
    def _():
        _p1(h1)

    plsc.subcore_barrier()

    def _p2(xe_ref):
        def blk(i, carry):
            base = s * ER_T + i * IB
            pltpu.sync_copy(acc.at[pl.ds(base, IB)], dbuf)
            pltpu.sync_copy(rce.at[pl.ds(base, IB)], rbuf)

            def rowstep(r, carry2):
                spl = rbuf[r]
                for k in range(DH // 16):
                    dbuf[r, pl.ds(k * 16, 16)] = (
                        dbuf[r, pl.ds(k * 16, 16)] * spl)
                return carry2
            lax.fori_loop(0, IB, rowstep, 0)
            pltpu.sync_copy(dbuf, xe_ref.at[pl.ds(base, IB)])
            return carry
        lax.fori_loop(0, ER_T // IB, blk, 0)

    @pl.when(c == 0)
    def _():
        _p2(xe0)

    @pl.when(c == 1)
    def _():
        _p2(xe1)

    plsc.subcore_barrier()

    _zero_rows()

    def _zv(i, carry):
        pltpu.sync_copy(rows, acc.at[pl.ds(s * VR_T + i * IB, IB)])
        return carry
    lax.fori_loop(0, VR_T // IB, _zv, 0)
    plsc.subcore_barrier()

    def _p3(xe_ref):
        def grp(g, carry):
            base = s * R_T + g * CH
            pltpu.sync_copy(vidx.at[pl.ds(base, CH)], vbuf)
            pltpu.sync_copy(eidx.at[pl.ds(base, CH)], ebuf)

            def step(j, carry2):
                pltpu.sync_copy(xe_ref.at[ebuf.at[j]], rows)
                pltpu.sync_copy(rows, acc.at[vbuf.at[j]], add=True)
                return carry2
            lax.fori_loop(0, CH, step, 0)
            return carry
        lax.fori_loop(0, R_T // CH, grp, 0)

    @pl.when(c == 0)
    def _():
        _p3(xe0)

    @pl.when(c == 1)
    def _():
        _p3(xe1)

    plsc.subcore_barrier()

    @pl.when(c == 0)
    def _():
        pltpu.sync_copy(acc.at[pl.ds(s * VR_T, VR_T)],
                        m0.at[pl.ds(s * VR_T, VR_T)])

    @pl.when(c == 1)
    def _():
        pltpu.sync_copy(acc.at[pl.ds(s * VR_T, VR_T)],
                        m1.at[pl.ds(s * VR_T, VR_T)])


@jax.jit
def _v2v(vidx2, eidx2, h0, h1, rce):
    return pl.kernel(
        _v2v_body,
        out_type=(
            jax.ShapeDtypeStruct((N_V, DH), _f32),
            jax.ShapeDtypeStruct((N_V, DH), _f32),
            jax.ShapeDtypeStruct((N_E, DH), _f32),
            jax.ShapeDtypeStruct((N_E, DH), _f32),
        ),
        mesh=plsc.VectorSubcoreMesh(core_axis_name="c", subcore_axis_name="s"),
        compiler_params=pltpu.CompilerParams(use_tc_tiling_on_sc=False),
        scratch_types=[
            pltpu.VMEM((CH, IB), _i32),
            pltpu.VMEM((CH, IB), _i32),
            pltpu.VMEM((IB, DH), _f32),
            pltpu.VMEM((IB, 16), _f32),
            pltpu.VMEM((IB, DH), _f32),
            pltpu.VMEM_SHARED((N_E, DH), _f32),
        ],
    )(vidx2, eidx2, h0, h1, rce)


_BM = 1000


def _mm1_body(x_ref, w_ref, b_ref, o0_ref, o1_ref):
    h = jnp.dot(x_ref[...], w_ref[...], preferred_element_type=_f32)
    h = jnp.maximum(h + b_ref[...], 0.0)
    o0_ref[...] = h[:, :DH]
    o1_ref[...] = h[:, DH:]


@jax.jit
def _mm1(x, w1, b1):
    return pl.pallas_call(
        _mm1_body,
        grid=(N_V // _BM,),
        in_specs=[
            pl.BlockSpec((_BM, D), lambda i: (i, 0)),
            pl.BlockSpec((D, D), lambda i: (0, 0)),
            pl.BlockSpec((1, D), lambda i: (0, 0)),
        ],
        out_specs=[
            pl.BlockSpec((_BM, DH), lambda i: (i, 0)),
            pl.BlockSpec((_BM, DH), lambda i: (i, 0)),
        ],
        out_shape=[
            jax.ShapeDtypeStruct((N_V, DH), _f32),
            jax.ShapeDtypeStruct((N_V, DH), _f32),
        ],
    )(x, w1, b1)


def _mm2_body(m0_ref, m1_ref, rv_ref, w_ref, b_ref, o0_ref, o1_ref):
    x = jnp.concatenate([m0_ref[...], m1_ref[...]], axis=1)
    x = jnp.maximum(x, 0.0) * rv_ref[...]
    h = jnp.dot(x, w_ref[...], preferred_element_type=_f32)
    h = jnp.maximum(h + b_ref[...], 0.0)
    o0_ref[...] = h[:, :DH]
    o1_ref[...] = h[:, DH:]


@jax.jit
def _mm2(m0, m1, rv, w2, b2):
    return pl.pallas_call(
        _mm2_body,
        grid=(N_V // _BM,),
        in_specs=[
            pl.BlockSpec((_BM, DH), lambda i: (i, 0)),
            pl.BlockSpec((_BM, DH), lambda i: (i, 0)),
            pl.BlockSpec((_BM, 1), lambda i: (i, 0)),
            pl.BlockSpec((D, D), lambda i: (0, 0)),
            pl.BlockSpec((1, D), lambda i: (0, 0)),
        ],
        out_specs=[
            pl.BlockSpec((_BM, DH), lambda i: (i, 0)),
            pl.BlockSpec((_BM, DH), lambda i: (i, 0)),
        ],
        out_shape=[
            jax.ShapeDtypeStruct((N_V, DH), _f32),
            jax.ShapeDtypeStruct((N_V, DH), _f32),
        ],
    )(m0, m1, rv, w2, b2)


def _merge_body(m0_ref, m1_ref, rv_ref, o_ref):
    x = jnp.concatenate([m0_ref[...], m1_ref[...]], axis=1)
    o_ref[...] = x * rv_ref[...]


@jax.jit
def _merge(m0, m1, rv):
    return pl.pallas_call(
        _merge_body,
        grid=(N_V // _BM,),
        in_specs=[
            pl.BlockSpec((_BM, DH), lambda i: (i, 0)),
            pl.BlockSpec((_BM, DH), lambda i: (i, 0)),
            pl.BlockSpec((_BM, 1), lambda i: (i, 0)),
        ],
        out_specs=pl.BlockSpec((_BM, D), lambda i: (i, 0)),
        out_shape=jax.ShapeDtypeStruct((N_V, D), _f32),
    )(m0, m1, rv)


def kernel(X, v_idx, e_idx, W1, b1, W2, b2):
    vidx2 = v_idx.astype(_i32).reshape(NROW, IB)
    eidx2 = e_idx.astype(_i32).reshape(NROW, IB)
    b1r = b1.reshape(1, D)
    b2r = b2.reshape(1, D)

    rce, rcv = _counts(vidx2, eidx2)
    rv = rcv[:N_V, :1]

    h0, h1 = _mm1(X, W1, b1r)
    m0, m1, _, _ = _v2v(vidx2, eidx2, h0, h1, rce)
    g0, g1 = _mm2(m0, m1, rv, W2, b2r)
    n0, n1, _, _ = _v2v(vidx2, eidx2, g0, g1, rce)
    return _merge(n0, n1, rv)


# speedup vs baseline: 7.3880x; 1.2482x over previous
import jax
import jax.numpy as jnp
from jax import lax
from jax.experimental import pallas as pl
from jax.experimental.pallas import tpu as pltpu
from jax.experimental.pallas import tpu_sc as plsc

N_V = 10000
N_E = 20000
NNZ = 320000
D = 128
DH = 64
NS = 16
IB = 125
NROW = NNZ // IB
R_T = NROW // NS
E_PAD = 20480
V_PAD = 10240
E_T = E_PAD // NS
V_T = V_PAD // NS
ER_T = N_E // NS
VR_T = N_V // NS
CH = 32

_f32 = jnp.float32
_i32 = jnp.int32


def _zeros16():
    return jnp.zeros((16,), _f32)


def _counts_body(vidx, eidx, rce_hbm, rcv_hbm, idx_buf, ones_buf, dbuf,
                 cnt_sh):
    c = lax.axis_index("c")
    s = lax.axis_index("s")

    def _z(i, carry):
        dbuf[i] = _zeros16()
        return carry
    lax.fori_loop(0, E_T, _z, 0)
    pltpu.sync_copy(dbuf, cnt_sh.at[pl.ds(s * E_T, E_T)])

    def _o(i, carry):
        ones_buf[i] = jnp.ones((16,), _f32)
        return carry
    lax.fori_loop(0, IB, _o, 0)

    @pl.when(c == 0)
    def _():
        pltpu.sync_copy(eidx.at[pl.ds(s * R_T, R_T)], idx_buf)

    @pl.when(c == 1)
    def _():
        pltpu.sync_copy(vidx.at[pl.ds(s * R_T, R_T)], idx_buf)

    plsc.subcore_barrier()

    def _sc(j, carry):
        pltpu.sync_copy(ones_buf, cnt_sh.at[idx_buf.at[j]], add=True)
        return carry
    lax.fori_loop(0, R_T, _sc, 0)

    plsc.subcore_barrier()

    def _extract(nrows, base, out_ref):
        pltpu.sync_copy(cnt_sh.at[pl.ds(base, nrows)], dbuf.at[pl.ds(0, nrows)])

        def _r(r, carry):
            dbuf[r] = 1.0 / jnp.maximum(dbuf[r], 1.0)
            return carry
        lax.fori_loop(0, nrows, _r, 0)
        pltpu.sync_copy(dbuf.at[pl.ds(0, nrows)], out_ref.at[pl.ds(base, nrows)])

    @pl.when(c == 0)
    def _():
        _extract(E_T, s * E_T, rce_hbm)

    @pl.when(c == 1)
    def _():
        _extract(V_T, s * V_T, rcv_hbm)


@jax.jit
def _counts(vidx2, eidx2):
    return pl.kernel(
        _counts_body,
        out_type=(
            jax.ShapeDtypeStruct((E_PAD, 16), _f32),
            jax.ShapeDtypeStruct((V_PAD, 16), _f32),
        ),
        mesh=plsc.VectorSubcoreMesh(core_axis_name="c", subcore_axis_name="s"),
        compiler_params=pltpu.CompilerParams(use_tc_tiling_on_sc=False),
        scratch_types=[
            pltpu.VMEM((R_T, IB), _i32),
            pltpu.VMEM((IB, 16), _f32),
            pltpu.VMEM((E_T, 16), _f32),
            pltpu.VMEM_SHARED((E_PAD, 16), _f32),
        ],
    )(vidx2, eidx2)


def _v2v_body(vidx, eidx, h0, h1, rce, m0, m1, xe0, xe1,
              vbuf, ebuf, rows, rows_b, rbuf, dbuf, acc,
              gsa, gsb, ssa, ssb):
    c = lax.axis_index("c")
    s = lax.axis_index("s")

    def _zero_rows():
        def _z(i, carry):
            for k in range(DH // 16):
                rows[i, pl.ds(k * 16, 16)] = _zeros16()
            return carry
        lax.fori_loop(0, IB, _z, 0)

    _zero_rows()

    def _ze(i, carry):
        pltpu.sync_copy(rows, acc.at[pl.ds(s * ER_T + i * IB, IB)])
        return carry
    lax.fori_loop(0, ER_T // IB, _ze, 0)
    plsc.subcore_barrier()

    def _hop(src_hbm, gbuf, sbuf):
        def grp(g, carry):
            base = s * R_T + g * CH
            pltpu.sync_copy(vidx.at[pl.ds(base, CH)], vbuf)
            pltpu.sync_copy(eidx.at[pl.ds(base, CH)], ebuf)
            pltpu.async_copy(src_hbm.at[gbuf.at[0]], rows, gsa)

            def pair(k, carry2):
                j0 = 2 * k

                @pl.when(k > 0)
                def _():
                    pltpu.make_async_copy(
                        rows_b, acc.at[sbuf.at[0]], ssb).wait()
                pltpu.make_async_copy(src_hbm.at[gbuf.at[0]], rows, gsa).wait()
                pltpu.async_copy(src_hbm.at[gbuf.at[j0 + 1]], rows_b, gsb)
                pltpu.async_copy(rows, acc.at[sbuf.at[j0]], ssa, add=True)
                pltpu.make_async_copy(
                    src_hbm.at[gbuf.at[0]], rows_b, gsb).wait()
                pltpu.async_copy(rows_b, acc.at[sbuf.at[j0 + 1]], ssb, add=True)
                pltpu.make_async_copy(rows, acc.at[sbuf.at[0]], ssa).wait()

                @pl.when(k < CH // 2 - 1)
                def _():
                    pltpu.async_copy(src_hbm.at[gbuf.at[j0 + 2]], rows, gsa)
                return carry2
            lax.fori_loop(0, CH // 2, pair, 0)
            pltpu.make_async_copy(rows_b, acc.at[sbuf.at[0]], ssb).wait()
            return carry
        lax.fori_loop(0, R_T // CH, grp, 0)

    def _p1(h_ref):
        _hop(h_ref, vbuf, ebuf)

    @pl.when(c == 0)
    def _():
        _p1(h0)

    @pl.when(c == 1)
    def _():
        _p1(h1)

    plsc.subcore_barrier()

    def _p2(xe_ref):
        def blk(i, carry):
            base = s * ER_T + i * IB
            pltpu.sync_copy(acc.at[pl.ds(base, IB)], dbuf)
            pltpu.sync_copy(rce.at[pl.ds(base, IB)], rbuf)

            def rowstep(r, carry2):
                spl = rbuf[r]
                for k in range(DH // 16):
                    dbuf[r, pl.ds(k * 16, 16)] = (
                        dbuf[r, pl.ds(k * 16, 16)] * spl)
                return carry2
            lax.fori_loop(0, IB, rowstep, 0)
            pltpu.sync_copy(dbuf, xe_ref.at[pl.ds(base, IB)])
            return carry
        lax.fori_loop(0, ER_T // IB, blk, 0)

    @pl.when(c == 0)
    def _():
        _p2(xe0)

    @pl.when(c == 1)
    def _():
        _p2(xe1)

    plsc.subcore_barrier()

    _zero_rows()

    def _zv(i, carry):
        pltpu.sync_copy(rows, acc.at[pl.ds(s * VR_T + i * IB, IB)])
        return carry
    lax.fori_loop(0, VR_T // IB, _zv, 0)
    plsc.subcore_barrier()

    def _p3(xe_ref):
        _hop(xe_ref, ebuf, vbuf)

    @pl.when(c == 0)
    def _():
        _p3(xe0)

    @pl.when(c == 1)
    def _():
        _p3(xe1)

    plsc.subcore_barrier()

    @pl.when(c == 0)
    def _():
        pltpu.sync_copy(acc.at[pl.ds(s * VR_T, VR_T)],
                        m0.at[pl.ds(s * VR_T, VR_T)])

    @pl.when(c == 1)
    def _():
        pltpu.sync_copy(acc.at[pl.ds(s * VR_T, VR_T)],
                        m1.at[pl.ds(s * VR_T, VR_T)])


@jax.jit
def _v2v(vidx2, eidx2, h0, h1, rce):
    return pl.kernel(
        _v2v_body,
        out_type=(
            jax.ShapeDtypeStruct((N_V, DH), _f32),
            jax.ShapeDtypeStruct((N_V, DH), _f32),
            jax.ShapeDtypeStruct((N_E, DH), _f32),
            jax.ShapeDtypeStruct((N_E, DH), _f32),
        ),
        mesh=plsc.VectorSubcoreMesh(core_axis_name="c", subcore_axis_name="s"),
        compiler_params=pltpu.CompilerParams(use_tc_tiling_on_sc=False),
        scratch_types=[
            pltpu.VMEM((CH, IB), _i32),
            pltpu.VMEM((CH, IB), _i32),
            pltpu.VMEM((IB, DH), _f32),
            pltpu.VMEM((IB, DH), _f32),
            pltpu.VMEM((IB, 16), _f32),
            pltpu.VMEM((IB, DH), _f32),
            pltpu.VMEM_SHARED((N_E, DH), _f32),
            pltpu.SemaphoreType.DMA,
            pltpu.SemaphoreType.DMA,
            pltpu.SemaphoreType.DMA,
            pltpu.SemaphoreType.DMA,
        ],
    )(vidx2, eidx2, h0, h1, rce)


_BM = 1000


def _mm1_body(x_ref, w_ref, b_ref, o0_ref, o1_ref):
    h = jnp.dot(x_ref[...], w_ref[...], preferred_element_type=_f32)
    h = jnp.maximum(h + b_ref[...], 0.0)
    o0_ref[...] = h[:, :DH]
    o1_ref[...] = h[:, DH:]


@jax.jit
def _mm1(x, w1, b1):
    return pl.pallas_call(
        _mm1_body,
        grid=(N_V // _BM,),
        in_specs=[
            pl.BlockSpec((_BM, D), lambda i: (i, 0)),
            pl.BlockSpec((D, D), lambda i: (0, 0)),
            pl.BlockSpec((1, D), lambda i: (0, 0)),
        ],
        out_specs=[
            pl.BlockSpec((_BM, DH), lambda i: (i, 0)),
            pl.BlockSpec((_BM, DH), lambda i: (i, 0)),
        ],
        out_shape=[
            jax.ShapeDtypeStruct((N_V, DH), _f32),
            jax.ShapeDtypeStruct((N_V, DH), _f32),
        ],
    )(x, w1, b1)


def _mm2_body(m0_ref, m1_ref, rv_ref, w_ref, b_ref, o0_ref, o1_ref):
    x = jnp.concatenate([m0_ref[...], m1_ref[...]], axis=1)
    x = jnp.maximum(x, 0.0) * rv_ref[...]
    h = jnp.dot(x, w_ref[...], preferred_element_type=_f32)
    h = jnp.maximum(h + b_ref[...], 0.0)
    o0_ref[...] = h[:, :DH]
    o1_ref[...] = h[:, DH:]


@jax.jit
def _mm2(m0, m1, rv, w2, b2):
    return pl.pallas_call(
        _mm2_body,
        grid=(N_V // _BM,),
        in_specs=[
            pl.BlockSpec((_BM, DH), lambda i: (i, 0)),
            pl.BlockSpec((_BM, DH), lambda i: (i, 0)),
            pl.BlockSpec((_BM, 1), lambda i: (i, 0)),
            pl.BlockSpec((D, D), lambda i: (0, 0)),
            pl.BlockSpec((1, D), lambda i: (0, 0)),
        ],
        out_specs=[
            pl.BlockSpec((_BM, DH), lambda i: (i, 0)),
            pl.BlockSpec((_BM, DH), lambda i: (i, 0)),
        ],
        out_shape=[
            jax.ShapeDtypeStruct((N_V, DH), _f32),
            jax.ShapeDtypeStruct((N_V, DH), _f32),
        ],
    )(m0, m1, rv, w2, b2)


def _merge_body(m0_ref, m1_ref, rv_ref, o_ref):
    x = jnp.concatenate([m0_ref[...], m1_ref[...]], axis=1)
    o_ref[...] = x * rv_ref[...]


@jax.jit
def _merge(m0, m1, rv):
    return pl.pallas_call(
        _merge_body,
        grid=(N_V // _BM,),
        in_specs=[
            pl.BlockSpec((_BM, DH), lambda i: (i, 0)),
            pl.BlockSpec((_BM, DH), lambda i: (i, 0)),
            pl.BlockSpec((_BM, 1), lambda i: (i, 0)),
        ],
        out_specs=pl.BlockSpec((_BM, D), lambda i: (i, 0)),
        out_shape=jax.ShapeDtypeStruct((N_V, D), _f32),
    )(m0, m1, rv)


def kernel(X, v_idx, e_idx, W1, b1, W2, b2):
    vidx2 = v_idx.astype(_i32).reshape(NROW, IB)
    eidx2 = e_idx.astype(_i32).reshape(NROW, IB)
    b1r = b1.reshape(1, D)
    b2r = b2.reshape(1, D)

    rce, rcv = _counts(vidx2, eidx2)
    rv = rcv[:N_V, :1]

    h0, h1 = _mm1(X, W1, b1r)
    m0, m1, _, _ = _v2v(vidx2, eidx2, h0, h1, rce)
    g0, g1 = _mm2(m0, m1, rv, W2, b2r)
    n0, n1, _, _ = _v2v(vidx2, eidx2, g0, g1, rce)
    return _merge(n0, n1, rv)


# speedup vs baseline: 10.7180x; 1.4507x over previous
import jax
import jax.numpy as jnp
from jax import lax
from jax.experimental import pallas as pl
from jax.experimental.pallas import tpu as pltpu
from jax.experimental.pallas import tpu_sc as plsc

N_V = 10000
N_E = 20000
NNZ = 320000
D = 128
DH = 64
NS = 16
IB = 125
NROW = NNZ // IB
R_T = NROW // NS
E_PAD = 20480
V_PAD = 10240
E_T = E_PAD // NS
V_T = V_PAD // NS
ER_T = N_E // NS
VR_T = N_V // NS
CH = 40

_f32 = jnp.float32
_i32 = jnp.int32


def _zeros16():
    return jnp.zeros((16,), _f32)


def _counts_body(vidx, eidx, rce_hbm, rcv_hbm, idx_buf, ones_buf, dbuf,
                 cnt_sh):
    c = lax.axis_index("c")
    s = lax.axis_index("s")

    def _z(i, carry):
        dbuf[i] = _zeros16()
        return carry
    lax.fori_loop(0, E_T, _z, 0)
    pltpu.sync_copy(dbuf, cnt_sh.at[pl.ds(s * E_T, E_T)])

    def _o(i, carry):
        ones_buf[i] = jnp.ones((16,), _f32)
        return carry
    lax.fori_loop(0, IB, _o, 0)

    @pl.when(c == 0)
    def _():
        pltpu.sync_copy(eidx.at[pl.ds(s * R_T, R_T)], idx_buf)

    @pl.when(c == 1)
    def _():
        pltpu.sync_copy(vidx.at[pl.ds(s * R_T, R_T)], idx_buf)

    plsc.subcore_barrier()

    def _sc(j, carry):
        pltpu.sync_copy(ones_buf, cnt_sh.at[idx_buf.at[j]], add=True)
        return carry
    lax.fori_loop(0, R_T, _sc, 0)

    plsc.subcore_barrier()

    def _extract(nrows, base, out_ref):
        pltpu.sync_copy(cnt_sh.at[pl.ds(base, nrows)], dbuf.at[pl.ds(0, nrows)])

        def _r(r, carry):
            dbuf[r] = 1.0 / jnp.maximum(dbuf[r], 1.0)
            return carry
        lax.fori_loop(0, nrows, _r, 0)
        pltpu.sync_copy(dbuf.at[pl.ds(0, nrows)], out_ref.at[pl.ds(base, nrows)])

    @pl.when(c == 0)
    def _():
        _extract(E_T, s * E_T, rce_hbm)

    @pl.when(c == 1)
    def _():
        _extract(V_T, s * V_T, rcv_hbm)


@jax.jit
def _counts(vidx2, eidx2):
    return pl.kernel(
        _counts_body,
        out_type=(
            jax.ShapeDtypeStruct((E_PAD, 16), _f32),
            jax.ShapeDtypeStruct((V_PAD, 16), _f32),
        ),
        mesh=plsc.VectorSubcoreMesh(core_axis_name="c", subcore_axis_name="s"),
        compiler_params=pltpu.CompilerParams(use_tc_tiling_on_sc=False),
        scratch_types=[
            pltpu.VMEM((R_T, IB), _i32),
            pltpu.VMEM((IB, 16), _f32),
            pltpu.VMEM((E_T, 16), _f32),
            pltpu.VMEM_SHARED((E_PAD, 16), _f32),
        ],
    )(vidx2, eidx2)


def _v2v_body(vidx, eidx, h0, h1, rce, m0, m1, xe0, xe1,
              vbuf, ebuf, r0, r1, r2, r3, rbuf, acc,
              g0, g1, g2, g3, s0, s1, s2, s3):
    c = lax.axis_index("c")
    s = lax.axis_index("s")
    bufs = (r0, r1, r2, r3)
    gsems = (g0, g1, g2, g3)
    ssems = (s0, s1, s2, s3)
    rows = r0

    def _zero_rows():
        def _z(i, carry):
            for k in range(DH // 16):
                rows[i, pl.ds(k * 16, 16)] = _zeros16()
            return carry
        lax.fori_loop(0, IB, _z, 0)

    _zero_rows()

    def _ze(i, carry):
        pltpu.sync_copy(rows, acc.at[pl.ds(s * ER_T + i * IB, IB)])
        return carry
    lax.fori_loop(0, ER_T // IB, _ze, 0)
    plsc.subcore_barrier()

    def _hop(src_hbm, gbuf, sbuf):
        def grp(g, carry):
            base = s * R_T + g * CH
            pltpu.sync_copy(vidx.at[pl.ds(base, CH)], vbuf)
            pltpu.sync_copy(eidx.at[pl.ds(base, CH)], ebuf)
            for i in range(3):
                pltpu.async_copy(src_hbm.at[gbuf.at[i]], bufs[i], gsems[i])

            def quad(k, carry2):
                for i in range(4):
                    j = 4 * k + i
                    pi = (i - 1) % 4
                    pltpu.make_async_copy(
                        src_hbm.at[gbuf.at[0]], bufs[i], gsems[i]).wait()
                    pltpu.async_copy(
                        bufs[i], acc.at[sbuf.at[j]], ssems[i], add=True)

                    @pl.when(j >= 1)
                    def _():
                        pltpu.make_async_copy(
                            bufs[pi], acc.at[sbuf.at[0]], ssems[pi]).wait()

                    @pl.when(j + 3 < CH)
                    def _():
                        pltpu.async_copy(
                            src_hbm.at[gbuf.at[j + 3]], bufs[pi], gsems[pi])
                return carry2
            lax.fori_loop(0, CH // 4, quad, 0)
            pltpu.make_async_copy(bufs[3], acc.at[sbuf.at[0]], ssems[3]).wait()
            return carry
        lax.fori_loop(0, R_T // CH, grp, 0)

    def _p1(h_ref):
        _hop(h_ref, vbuf, ebuf)

    @pl.when(c == 0)
    def _():
        _p1(h0)

    @pl.when(c == 1)
    def _():
        _p1(h1)

    plsc.subcore_barrier()

    def _p2(xe_ref):
        def blk(i, carry):
            base = s * ER_T + i * IB
            pltpu.sync_copy(acc.at[pl.ds(base, IB)], r1)
            pltpu.sync_copy(rce.at[pl.ds(base, IB)], rbuf)

            def rowstep(r, carry2):
                spl = rbuf[r]
                for k in range(DH // 16):
                    r1[r, pl.ds(k * 16, 16)] = (
                        r1[r, pl.ds(k * 16, 16)] * spl)
                return carry2
            lax.fori_loop(0, IB, rowstep, 0)
            pltpu.sync_copy(r1, xe_ref.at[pl.ds(base, IB)])
            return carry
        lax.fori_loop(0, ER_T // IB, blk, 0)

    @pl.when(c == 0)
    def _():
        _p2(xe0)

    @pl.when(c == 1)
    def _():
        _p2(xe1)

    plsc.subcore_barrier()

    _zero_rows()

    def _zv(i, carry):
        pltpu.sync_copy(rows, acc.at[pl.ds(s * VR_T + i * IB, IB)])
        return carry
    lax.fori_loop(0, VR_T // IB, _zv, 0)
    plsc.subcore_barrier()

    def _p3(xe_ref):
        _hop(xe_ref, ebuf, vbuf)

    @pl.when(c == 0)
    def _():
        _p3(xe0)

    @pl.when(c == 1)
    def _():
        _p3(xe1)

    plsc.subcore_barrier()

    @pl.when(c == 0)
    def _():
        pltpu.sync_copy(acc.at[pl.ds(s * VR_T, VR_T)],
                        m0.at[pl.ds(s * VR_T, VR_T)])

    @pl.when(c == 1)
    def _():
        pltpu.sync_copy(acc.at[pl.ds(s * VR_T, VR_T)],
                        m1.at[pl.ds(s * VR_T, VR_T)])


@jax.jit
def _v2v(vidx2, eidx2, h0, h1, rce):
    return pl.kernel(
        _v2v_body,
        out_type=(
            jax.ShapeDtypeStruct((N_V, DH), _f32),
            jax.ShapeDtypeStruct((N_V, DH), _f32),
            jax.ShapeDtypeStruct((N_E, DH), _f32),
            jax.ShapeDtypeStruct((N_E, DH), _f32),
        ),
        mesh=plsc.VectorSubcoreMesh(core_axis_name="c", subcore_axis_name="s"),
        compiler_params=pltpu.CompilerParams(use_tc_tiling_on_sc=False),
        scratch_types=[
            pltpu.VMEM((CH, IB), _i32),
            pltpu.VMEM((CH, IB), _i32),
            pltpu.VMEM((IB, DH), _f32),
            pltpu.VMEM((IB, DH), _f32),
            pltpu.VMEM((IB, DH), _f32),
            pltpu.VMEM((IB, DH), _f32),
            pltpu.VMEM((IB, 16), _f32),
            pltpu.VMEM_SHARED((N_E, DH), _f32),
        ] + [pltpu.SemaphoreType.DMA] * 8,
    )(vidx2, eidx2, h0, h1, rce)


_BM = 1000


def _mm1_body(x_ref, w_ref, b_ref, o0_ref, o1_ref):
    h = jnp.dot(x_ref[...], w_ref[...], preferred_element_type=_f32)
    h = jnp.maximum(h + b_ref[...], 0.0)
    o0_ref[...] = h[:, :DH]
    o1_ref[...] = h[:, DH:]


@jax.jit
def _mm1(x, w1, b1):
    return pl.pallas_call(
        _mm1_body,
        grid=(N_V // _BM,),
        in_specs=[
            pl.BlockSpec((_BM, D), lambda i: (i, 0)),
            pl.BlockSpec((D, D), lambda i: (0, 0)),
            pl.BlockSpec((1, D), lambda i: (0, 0)),
        ],
        out_specs=[
            pl.BlockSpec((_BM, DH), lambda i: (i, 0)),
            pl.BlockSpec((_BM, DH), lambda i: (i, 0)),
        ],
        out_shape=[
            jax.ShapeDtypeStruct((N_V, DH), _f32),
            jax.ShapeDtypeStruct((N_V, DH), _f32),
        ],
    )(x, w1, b1)


def _mm2_body(m0_ref, m1_ref, rv_ref, w_ref, b_ref, o0_ref, o1_ref):
    x = jnp.concatenate([m0_ref[...], m1_ref[...]], axis=1)
    x = jnp.maximum(x, 0.0) * rv_ref[...]
    h = jnp.dot(x, w_ref[...], preferred_element_type=_f32)
    h = jnp.maximum(h + b_ref[...], 0.0)
    o0_ref[...] = h[:, :DH]
    o1_ref[...] = h[:, DH:]


@jax.jit
def _mm2(m0, m1, rv, w2, b2):
    return pl.pallas_call(
        _mm2_body,
        grid=(N_V // _BM,),
        in_specs=[
            pl.BlockSpec((_BM, DH), lambda i: (i, 0)),
            pl.BlockSpec((_BM, DH), lambda i: (i, 0)),
            pl.BlockSpec((_BM, 1), lambda i: (i, 0)),
            pl.BlockSpec((D, D), lambda i: (0, 0)),
            pl.BlockSpec((1, D), lambda i: (0, 0)),
        ],
        out_specs=[
            pl.BlockSpec((_BM, DH), lambda i: (i, 0)),
            pl.BlockSpec((_BM, DH), lambda i: (i, 0)),
        ],
        out_shape=[
            jax.ShapeDtypeStruct((N_V, DH), _f32),
            jax.ShapeDtypeStruct((N_V, DH), _f32),
        ],
    )(m0, m1, rv, w2, b2)


def _merge_body(m0_ref, m1_ref, rv_ref, o_ref):
    x = jnp.concatenate([m0_ref[...], m1_ref[...]], axis=1)
    o_ref[...] = x * rv_ref[...]


@jax.jit
def _merge(m0, m1, rv):
    return pl.pallas_call(
        _merge_body,
        grid=(N_V // _BM,),
        in_specs=[
            pl.BlockSpec((_BM, DH), lambda i: (i, 0)),
            pl.BlockSpec((_BM, DH), lambda i: (i, 0)),
            pl.BlockSpec((_BM, 1), lambda i: (i, 0)),
        ],
        out_specs=pl.BlockSpec((_BM, D), lambda i: (i, 0)),
        out_shape=jax.ShapeDtypeStruct((N_V, D), _f32),
    )(m0, m1, rv)


def kernel(X, v_idx, e_idx, W1, b1, W2, b2):
    vidx2 = v_idx.astype(_i32).reshape(NROW, IB)
    eidx2 = e_idx.astype(_i32).reshape(NROW, IB)
    b1r = b1.reshape(1, D)
    b2r = b2.reshape(1, D)

    rce, rcv = _counts(vidx2, eidx2)
    rv = rcv[:N_V, :1]

    h0, h1 = _mm1(X, W1, b1r)
    m0, m1, _, _ = _v2v(vidx2, eidx2, h0, h1, rce)
    g0, g1 = _mm2(m0, m1, rv, W2, b2r)
    n0, n1, _, _ = _v2v(vidx2, eidx2, g0, g1, rce)
    return _merge(n0, n1, rv)
